# baseline (speedup 1.0000x reference)
import functools

import jax
import jax.numpy as jnp
from jax import lax
from jax.experimental import pallas as pl
from jax.experimental.pallas import tpu as pltpu
from jax.experimental.pallas import tpu_sc as plsc

E = 64
TOP_K = 8
N_GROUP = 16
PER_GROUP = E // N_GROUP
TOPK_GROUP = 4
SCALE = 2.5

TB = 1024
NW = 32
CH = 16


def _score_kernel(x_ref, w_ref, b_ref, sfc_ref):
    x = x_ref[...]
    w = w_ref[...]
    logits_tok = lax.dot_general(
        x, w, (((1,), (1,)), ((), ())), preferred_element_type=jnp.float32
    )
    logits = logits_tok.T
    sfc_ref[...] = jax.nn.sigmoid(logits) + b_ref[...]


def _route_kernel(sfc_hbm, bias_hbm, idx_hbm, wt_hbm, sfc_v, bias_v, idx_v, wt_v, sem):
    t = sfc_hbm.shape[1]
    nwords = sfc_v.shape[0]
    tpw = nwords // E
    wid = lax.axis_index("s") * 2 + lax.axis_index("c")
    base = wid * tpw

    cps = [
        pltpu.async_copy(
            sfc_hbm.at[e, pl.ds(base, tpw)], sfc_v.at[pl.ds(e * tpw, tpw)], sem
        )
        for e in range(E)
    ]
    cps.append(pltpu.async_copy(bias_hbm, bias_v, sem))
    for cp in cps:
        cp.wait()

    lanes = lax.iota(jnp.int32, CH)
    neg = jnp.full((CH,), -jnp.inf, jnp.float32)

    def chunk(c, carry):
        off = c * CH
        tok = off + lanes
        v = [sfc_v[pl.ds(e * tpw + off, CH)] for e in range(E)]

        gs = []
        for g in range(N_GROUP):
            e0, e1, e2, e3 = v[4 * g : 4 * g + 4]
            m = jnp.maximum(e0 + e1, e0 + e2)
            m = jnp.maximum(m, e0 + e3)
            m = jnp.maximum(m, e1 + e2)
            m = jnp.maximum(m, e1 + e3)
            m = jnp.maximum(m, e2 + e3)
            gs.append(m)

        gids = []
        for _ in range(TOPK_GROUP):
            gmax = gs[0]
            for g in range(1, N_GROUP):
                gmax = jnp.maximum(gmax, gs[g])
            gid = jnp.zeros((CH,), jnp.int32)
            for g in range(N_GROUP - 1, -1, -1):
                gid = jnp.where(gs[g] == gmax, g, gid)
            for g in range(N_GROUP):
                gs[g] = jnp.where(gid == g, neg, gs[g])
            gids.append(gid)

        svals, sidx = [], []
        for k in range(TOPK_GROUP):
            for j in range(PER_GROUP):
                eidx = gids[k] * PER_GROUP + j
                svals.append(plsc.load_gather(sfc_v, [eidx * tpw + tok]))
                sidx.append(eidx)

        wsel = []
        dsum = jnp.zeros((CH,), jnp.float32)
        for i in range(TOP_K):
            bv, bi = svals[0], sidx[0]
            for s in range(1, 16):
                cond = (svals[s] > bv) | ((svals[s] == bv) & (sidx[s] < bi))
                bv = jnp.where(cond, svals[s], bv)
                bi = jnp.where(cond, sidx[s], bi)
            for s in range(16):
                svals[s] = jnp.where(sidx[s] == bi, neg, svals[s])
            w = bv - plsc.load_gather(bias_v, [bi])
            idx_v[pl.ds(i * tpw + off, CH)] = bi
            wsel.append(w)
            dsum = dsum + w

        r = SCALE / (dsum + 1e-20)
        for i in range(TOP_K):
            wt_v[pl.ds(i * tpw + off, CH)] = wsel[i] * r
        return carry

    lax.fori_loop(0, tpw // CH, chunk, 0)

    ocps = [
        pltpu.async_copy(
            idx_v.at[pl.ds(i * tpw, tpw)], idx_hbm.at[i, pl.ds(base, tpw)], sem
        )
        for i in range(TOP_K)
    ]
    ocps += [
        pltpu.async_copy(
            wt_v.at[pl.ds(i * tpw, tpw)], wt_hbm.at[i, pl.ds(base, tpw)], sem
        )
        for i in range(TOP_K)
    ]
    for cp in ocps:
        cp.wait()


NSLICE = 2


@functools.partial(jax.jit, static_argnames=())
def kernel(hidden_states, weight, e_score_correction_bias):
    bsz, seq_len, h = hidden_states.shape
    t = bsz * seq_len
    ts = t // NSLICE
    tpw = ts // NW
    x = hidden_states.reshape(t, h).astype(jnp.float32)
    w32 = weight.astype(jnp.float32)
    bias1d = e_score_correction_bias.astype(jnp.float32)
    bias2d = bias1d.reshape(E, 1)

    score = functools.partial(
        pl.pallas_call,
        _score_kernel,
        grid=(ts // TB,),
        in_specs=[
            pl.BlockSpec((TB, h), lambda i: (i, 0)),
            pl.BlockSpec((E, h), lambda i: (0, 0)),
            pl.BlockSpec((E, 1), lambda i: (0, 0)),
        ],
        out_specs=pl.BlockSpec((E, TB), lambda i: (0, i)),
        out_shape=jax.ShapeDtypeStruct((E, ts), jnp.float32),
    )()
    route = functools.partial(
        pl.kernel,
        mesh=plsc.VectorSubcoreMesh(core_axis_name="c", subcore_axis_name="s"),
        compiler_params=pltpu.CompilerParams(needs_layout_passes=False),
        out_type=[
            jax.ShapeDtypeStruct((TOP_K, ts), jnp.int32),
            jax.ShapeDtypeStruct((TOP_K, ts), jnp.float32),
        ],
        scratch_types=[
            pltpu.VMEM((E * tpw,), jnp.float32),
            pltpu.VMEM((E,), jnp.float32),
            pltpu.VMEM((TOP_K * tpw,), jnp.int32),
            pltpu.VMEM((TOP_K * tpw,), jnp.float32),
            pltpu.SemaphoreType.DMA,
        ],
    )(_route_kernel)

    sfcs = [score(x[s * ts : (s + 1) * ts], w32, bias2d) for s in range(NSLICE)]
    routed = [route(sfc, bias1d) for sfc in sfcs]
    idx_t = jnp.concatenate([r[0] for r in routed], axis=1)
    wt_t = jnp.concatenate([r[1] for r in routed], axis=1)
    return idx_t.T, wt_t.T


# speedup vs baseline: 2.0207x; 2.0207x over previous
import functools

import jax
import jax.numpy as jnp
from jax import lax
from jax.experimental import pallas as pl
from jax.experimental.pallas import tpu as pltpu
from jax.experimental.pallas import tpu_sc as plsc

E = 64
TOP_K = 8
N_GROUP = 16
PER_GROUP = E // N_GROUP
TOPK_GROUP = 4
SCALE = 2.5

TB = 1024
NW = 32
CH = 16


def _score_kernel(x_ref, w_ref, b_ref, sfc_ref):
    x = x_ref[...]
    w = w_ref[...]
    logits_tok = lax.dot_general(
        x, w, (((1,), (1,)), ((), ())), preferred_element_type=jnp.float32
    )
    logits = logits_tok.T
    sfc_ref[...] = jax.nn.sigmoid(logits) + b_ref[...]


def _route_kernel(sfc_hbm, bias_hbm, idx_hbm, wt_hbm, sfc_v, bias_v, idx_v, wt_v, sem):
    t = sfc_hbm.shape[1]
    nwords = sfc_v.shape[0]
    tpw = nwords // E
    wid = lax.axis_index("s") * 2 + lax.axis_index("c")
    base = wid * tpw

    cps = [
        pltpu.async_copy(
            sfc_hbm.at[e, pl.ds(base, tpw)], sfc_v.at[pl.ds(e * tpw, tpw)], sem
        )
        for e in range(E)
    ]
    cps.append(pltpu.async_copy(bias_hbm, bias_v, sem))
    for cp in cps:
        cp.wait()

    lanes = lax.iota(jnp.int32, CH)
    neg = jnp.full((CH,), -jnp.inf, jnp.float32)

    def chunk(c, carry):
        off = c * CH
        tok = off + lanes
        v = [sfc_v[pl.ds(e * tpw + off, CH)] for e in range(E)]

        gs = []
        for g in range(N_GROUP):
            e0, e1, e2, e3 = v[4 * g : 4 * g + 4]
            m = jnp.maximum(e0 + e1, e0 + e2)
            m = jnp.maximum(m, e0 + e3)
            m = jnp.maximum(m, e1 + e2)
            m = jnp.maximum(m, e1 + e3)
            m = jnp.maximum(m, e2 + e3)
            gs.append(m)

        gids = []
        for _ in range(TOPK_GROUP):
            gmax = gs[0]
            for g in range(1, N_GROUP):
                gmax = jnp.maximum(gmax, gs[g])
            gid = jnp.zeros((CH,), jnp.int32)
            for g in range(N_GROUP - 1, -1, -1):
                gid = jnp.where(gs[g] == gmax, g, gid)
            for g in range(N_GROUP):
                gs[g] = jnp.where(gid == g, neg, gs[g])
            gids.append(gid)

        svals, sidx = [], []
        for k in range(TOPK_GROUP):
            for j in range(PER_GROUP):
                eidx = gids[k] * PER_GROUP + j
                svals.append(plsc.load_gather(sfc_v, [eidx * tpw + tok]))
                sidx.append(eidx)

        wsel = []
        dsum = jnp.zeros((CH,), jnp.float32)
        for i in range(TOP_K):
            bv, bi = svals[0], sidx[0]
            for s in range(1, 16):
                cond = (svals[s] > bv) | ((svals[s] == bv) & (sidx[s] < bi))
                bv = jnp.where(cond, svals[s], bv)
                bi = jnp.where(cond, sidx[s], bi)
            for s in range(16):
                svals[s] = jnp.where(sidx[s] == bi, neg, svals[s])
            w = bv - plsc.load_gather(bias_v, [bi])
            idx_v[pl.ds(i * tpw + off, CH)] = bi
            wsel.append(w)
            dsum = dsum + w

        r = SCALE / (dsum + 1e-20)
        for i in range(TOP_K):
            wt_v[pl.ds(i * tpw + off, CH)] = wsel[i] * r
        return carry

    lax.fori_loop(0, tpw // CH, chunk, 0)

    ocps = [
        pltpu.async_copy(
            idx_v.at[pl.ds(i * tpw, tpw)], idx_hbm.at[i, pl.ds(base, tpw)], sem
        )
        for i in range(TOP_K)
    ]
    ocps += [
        pltpu.async_copy(
            wt_v.at[pl.ds(i * tpw, tpw)], wt_hbm.at[i, pl.ds(base, tpw)], sem
        )
        for i in range(TOP_K)
    ]
    for cp in ocps:
        cp.wait()


NSLICE = 2


@functools.partial(jax.jit, static_argnames=())
def kernel(hidden_states, weight, e_score_correction_bias):
    bsz, seq_len, h = hidden_states.shape
    t = bsz * seq_len
    ts = t // NSLICE
    tpw = ts // NW
    x = hidden_states.reshape(t, h).astype(jnp.float32)
    w32 = weight.astype(jnp.float32)
    bias1d = e_score_correction_bias.astype(jnp.float32)
    bias2d = bias1d.reshape(E, 1)

    def score(s):
        blk0 = s * (ts // TB)
        return pl.pallas_call(
            _score_kernel,
            grid=(ts // TB,),
            in_specs=[
                pl.BlockSpec((TB, h), lambda i: (blk0 + i, 0)),
                pl.BlockSpec((E, h), lambda i: (0, 0)),
                pl.BlockSpec((E, 1), lambda i: (0, 0)),
            ],
            out_specs=pl.BlockSpec((E, TB), lambda i: (0, i)),
            out_shape=jax.ShapeDtypeStruct((E, ts), jnp.float32),
        )
    route = functools.partial(
        pl.kernel,
        mesh=plsc.VectorSubcoreMesh(core_axis_name="c", subcore_axis_name="s"),
        compiler_params=pltpu.CompilerParams(needs_layout_passes=False),
        out_type=[
            jax.ShapeDtypeStruct((TOP_K, ts), jnp.int32),
            jax.ShapeDtypeStruct((TOP_K, ts), jnp.float32),
        ],
        scratch_types=[
            pltpu.VMEM((E * tpw,), jnp.float32),
            pltpu.VMEM((E,), jnp.float32),
            pltpu.VMEM((TOP_K * tpw,), jnp.int32),
            pltpu.VMEM((TOP_K * tpw,), jnp.float32),
            pltpu.SemaphoreType.DMA,
        ],
    )(_route_kernel)

    sfcs = [score(s)(x, w32, bias2d) for s in range(NSLICE)]
    routed = [route(sfc, bias1d) for sfc in sfcs]
    idx_t = jnp.concatenate([r[0] for r in routed], axis=1)
    wt_t = jnp.concatenate([r[1] for r in routed], axis=1)
    return idx_t.T, wt_t.T


# speedup vs baseline: 2.2473x; 1.1122x over previous
import functools

import jax
import jax.numpy as jnp
from jax import lax
from jax.experimental import pallas as pl
from jax.experimental.pallas import tpu as pltpu
from jax.experimental.pallas import tpu_sc as plsc

E = 64
TOP_K = 8
N_GROUP = 16
PER_GROUP = E // N_GROUP
TOPK_GROUP = 4
SCALE = 2.5

TB = 1024
NW = 32
CH = 16


def _score_kernel(x_ref, w_ref, b_ref, sfc_ref):
    x = x_ref[...]
    w = w_ref[...]
    logits_tok = lax.dot_general(
        x, w, (((1,), (1,)), ((), ())), preferred_element_type=jnp.float32
    )
    logits = logits_tok.T
    sfc_ref[...] = jax.nn.sigmoid(logits) + b_ref[...]


def _route_kernel(sfc_hbm, bias_hbm, idx_hbm, wt_hbm, sfc_v, bias_v, idx_v, wt_v, sem):
    t = sfc_hbm.shape[1]
    nwords = sfc_v.shape[0]
    tpw = nwords // E
    wid = lax.axis_index("s") * 2 + lax.axis_index("c")
    base = wid * tpw

    cps = [
        pltpu.async_copy(
            sfc_hbm.at[e, pl.ds(base, tpw)], sfc_v.at[pl.ds(e * tpw, tpw)], sem
        )
        for e in range(E)
    ]
    cps.append(pltpu.async_copy(bias_hbm, bias_v, sem))
    for cp in cps:
        cp.wait()

    lanes = lax.iota(jnp.int32, CH)
    neg = jnp.full((CH,), -jnp.inf, jnp.float32)

    def chunk(c, carry):
        off = c * CH
        tok = off + lanes
        v = [sfc_v[pl.ds(e * tpw + off, CH)] for e in range(E)]

        gs = []
        for g in range(N_GROUP):
            e0, e1, e2, e3 = v[4 * g : 4 * g + 4]
            m = jnp.maximum(e0 + e1, e0 + e2)
            m = jnp.maximum(m, e0 + e3)
            m = jnp.maximum(m, e1 + e2)
            m = jnp.maximum(m, e1 + e3)
            m = jnp.maximum(m, e2 + e3)
            gs.append(m)

        gids = []
        for _ in range(TOPK_GROUP):
            mt = list(gs)
            while len(mt) > 1:
                mt = [
                    jnp.maximum(mt[2 * a], mt[2 * a + 1])
                    for a in range(len(mt) // 2)
                ]
            gmax = mt[0]
            cand = [
                jnp.where(gs[g] == gmax, g, N_GROUP) for g in range(N_GROUP)
            ]
            while len(cand) > 1:
                cand = [
                    jnp.minimum(cand[2 * a], cand[2 * a + 1])
                    for a in range(len(cand) // 2)
                ]
            gid = cand[0]
            for g in range(N_GROUP):
                gs[g] = jnp.where(gid == g, neg, gs[g])
            gids.append(gid)

        svals, sidx = [], []
        for k in range(TOPK_GROUP):
            for j in range(PER_GROUP):
                eidx = gids[k] * PER_GROUP + j
                svals.append(plsc.load_gather(sfc_v, [eidx * tpw + tok]))
                sidx.append(eidx)

        wsel = []
        dsum = jnp.zeros((CH,), jnp.float32)
        for i in range(TOP_K):
            tv, ti = list(svals), list(sidx)
            while len(tv) > 1:
                nv, ni = [], []
                for a in range(len(tv) // 2):
                    va, ia = tv[2 * a], ti[2 * a]
                    vb, ib = tv[2 * a + 1], ti[2 * a + 1]
                    cond = (vb > va) | ((vb == va) & (ib < ia))
                    nv.append(jnp.where(cond, vb, va))
                    ni.append(jnp.where(cond, ib, ia))
                tv, ti = nv, ni
            bv, bi = tv[0], ti[0]
            for s in range(16):
                svals[s] = jnp.where(sidx[s] == bi, neg, svals[s])
            w = bv - plsc.load_gather(bias_v, [bi])
            idx_v[pl.ds(i * tpw + off, CH)] = bi
            wsel.append(w)
            dsum = dsum + w

        r = SCALE / (dsum + 1e-20)
        for i in range(TOP_K):
            wt_v[pl.ds(i * tpw + off, CH)] = wsel[i] * r
        return carry

    lax.fori_loop(0, tpw // CH, chunk, 0)

    ocps = [
        pltpu.async_copy(
            idx_v.at[pl.ds(i * tpw, tpw)], idx_hbm.at[i, pl.ds(base, tpw)], sem
        )
        for i in range(TOP_K)
    ]
    ocps += [
        pltpu.async_copy(
            wt_v.at[pl.ds(i * tpw, tpw)], wt_hbm.at[i, pl.ds(base, tpw)], sem
        )
        for i in range(TOP_K)
    ]
    for cp in ocps:
        cp.wait()


@functools.partial(jax.jit, static_argnames=())
def kernel(hidden_states, weight, e_score_correction_bias):
    bsz, seq_len, h = hidden_states.shape
    t = bsz * seq_len
    tpw = t // NW
    x = hidden_states.reshape(t, h).astype(jnp.float32)
    bias1d = e_score_correction_bias.astype(jnp.float32)
    bias2d = bias1d.reshape(E, 1)

    sfc_t = pl.pallas_call(
        _score_kernel,
        grid=(t // TB,),
        in_specs=[
            pl.BlockSpec((TB, h), lambda i: (i, 0)),
            pl.BlockSpec((E, h), lambda i: (0, 0)),
            pl.BlockSpec((E, 1), lambda i: (0, 0)),
        ],
        out_specs=pl.BlockSpec((E, TB), lambda i: (0, i)),
        out_shape=jax.ShapeDtypeStruct((E, t), jnp.float32),
    )(x, weight.astype(jnp.float32), bias2d)

    route = functools.partial(
        pl.kernel,
        mesh=plsc.VectorSubcoreMesh(core_axis_name="c", subcore_axis_name="s"),
        compiler_params=pltpu.CompilerParams(needs_layout_passes=False),
        out_type=[
            jax.ShapeDtypeStruct((TOP_K, t), jnp.int32),
            jax.ShapeDtypeStruct((TOP_K, t), jnp.float32),
        ],
        scratch_types=[
            pltpu.VMEM((E * tpw,), jnp.float32),
            pltpu.VMEM((E,), jnp.float32),
            pltpu.VMEM((TOP_K * tpw,), jnp.int32),
            pltpu.VMEM((TOP_K * tpw,), jnp.float32),
            pltpu.SemaphoreType.DMA,
        ],
    )(_route_kernel)
    idx_t, wt_t = route(sfc_t, bias1d)
    return idx_t.T, wt_t.T
